# Initial kernel scaffold; baseline (speedup 1.0000x reference)
import jax
import jax.numpy as jnp
from jax.experimental import pallas as pl


def kernel(x, weight):
    raise NotImplementedError("write your pallas kernel here")


# speedup vs baseline: 1.8438x; 1.8438x over previous
import functools

import jax
import jax.numpy as jnp
from jax import lax
from jax.experimental import pallas as pl
from jax.experimental.pallas import tpu as pltpu
from jax.experimental.pallas import tpu_sc as plsc

N_VOCAB = 1000000
N_STATE = 64
BATCH = 16384
HIST = 50

NC = 2
NS = 16
NW = NC * NS

N_FLAT = BATCH * HIST
G = 128
K = 8
GROUPS_TOTAL = N_FLAT // G
GROUPS_PER_W = GROUPS_TOTAL // NW
NB = GROUPS_PER_W // K


@functools.partial(
    pl.kernel,
    out_type=jax.ShapeDtypeStruct((N_FLAT, N_STATE), jnp.float32),
    mesh=plsc.VectorSubcoreMesh(core_axis_name="c", subcore_axis_name="s"),
    scratch_types=[
        pltpu.VMEM((K, G), jnp.int32),
        pltpu.VMEM((K * G, N_STATE), jnp.float32),
        pltpu.SemaphoreType.DMA,
    ],
    compiler_params=pltpu.CompilerParams(use_tc_tiling_on_sc=False),
)
def _emb_lookup(idx_hbm, table_hbm, out_hbm, idx_v, rows_v, sem):
    wid = lax.axis_index("s") * NC + lax.axis_index("c")
    g0 = wid * GROUPS_PER_W

    def block_body(b, carry):
        g = g0 + b * K
        pltpu.sync_copy(idx_hbm.at[pl.ds(g, K)], idx_v)
        copies = []
        for j in range(K):
            copies.append(
                pltpu.async_copy(
                    table_hbm.at[idx_v.at[j]],
                    rows_v.at[pl.ds(j * G, G)],
                    sem,
                )
            )
        for c in copies:
            c.wait()
        pltpu.sync_copy(rows_v, out_hbm.at[pl.ds(g * G, K * G)])
        return carry

    lax.fori_loop(0, NB, block_body, 0)


def kernel(x, weight):
    idx = x.reshape(GROUPS_TOTAL, G).astype(jnp.int32)
    out = _emb_lookup(idx, weight)
    return out.reshape(BATCH, HIST, N_STATE)


# speedup vs baseline: 1.8792x; 1.0192x over previous
import functools

import jax
import jax.numpy as jnp
from jax import lax
from jax.experimental import pallas as pl
from jax.experimental.pallas import tpu as pltpu
from jax.experimental.pallas import tpu_sc as plsc

N_VOCAB = 1000000
N_STATE = 64
BATCH = 16384
HIST = 50

NC = 2
NS = 16
NW = NC * NS

N_FLAT = BATCH * HIST
G = 128
K = 4
GROUPS_TOTAL = N_FLAT // G
GROUPS_PER_W = GROUPS_TOTAL // NW
NB = GROUPS_PER_W // K


@functools.partial(
    pl.kernel,
    out_type=jax.ShapeDtypeStruct((N_FLAT, N_STATE), jnp.float32),
    mesh=plsc.VectorSubcoreMesh(core_axis_name="c", subcore_axis_name="s"),
    scratch_types=[
        pltpu.VMEM((2, K, G), jnp.int32),
        pltpu.VMEM((2, K * G, N_STATE), jnp.float32),
        pltpu.SemaphoreType.DMA,
        pltpu.SemaphoreType.DMA,
        pltpu.SemaphoreType.DMA,
        pltpu.SemaphoreType.DMA,
    ],
    compiler_params=pltpu.CompilerParams(use_tc_tiling_on_sc=False),
)
def _emb_lookup(idx_hbm, table_hbm, out_hbm, idx_v, rows_v, g0sem, g1sem, w0sem, w1sem):
    wid = lax.axis_index("s") * NC + lax.axis_index("c")
    g0 = wid * GROUPS_PER_W
    gsem = (g0sem, g1sem)
    wsem = (w0sem, w1sem)

    def load_idx(t, b):
        pltpu.sync_copy(idx_hbm.at[pl.ds(g0 + t * K, K)], idx_v.at[b])

    def fire_gathers(b):
        for j in range(K):
            pltpu.async_copy(
                table_hbm.at[idx_v.at[b, j]],
                rows_v.at[b, pl.ds(j * G, G)],
                gsem[b],
            )

    def drain_gathers(b):
        pltpu.make_async_copy(out_hbm.at[pl.ds(0, K * G)], rows_v.at[b], gsem[b]).wait()

    def fire_wb(t, b):
        pltpu.async_copy(rows_v.at[b], out_hbm.at[pl.ds((g0 + t * K) * G, K * G)], wsem[b])

    def wait_wb(b):
        pltpu.make_async_copy(rows_v.at[b], out_hbm.at[pl.ds(0, K * G)], wsem[b]).wait()

    load_idx(0, 0)
    fire_gathers(0)
    load_idx(1, 1)
    fire_gathers(1)
    drain_gathers(0)
    fire_wb(0, 0)

    @pl.loop(2, NB, step=2)
    def _steady(t):
        for b in range(2):
            tb = t + b
            load_idx(tb, b)
            wait_wb(b)
            fire_gathers(b)
            drain_gathers(1 - b)
            fire_wb(tb - 1, 1 - b)

    drain_gathers(1)
    fire_wb(NB - 1, 1)
    wait_wb(0)
    wait_wb(1)


def kernel(x, weight):
    idx = x.reshape(GROUPS_TOTAL, G).astype(jnp.int32)
    out = _emb_lookup(idx, weight)
    return out.reshape(BATCH, HIST, N_STATE)


# speedup vs baseline: 1.8851x; 1.0031x over previous
import functools

import jax
import jax.numpy as jnp
from jax import lax
from jax.experimental import pallas as pl
from jax.experimental.pallas import tpu as pltpu
from jax.experimental.pallas import tpu_sc as plsc

N_VOCAB = 1000000
N_STATE = 64
BATCH = 16384
HIST = 50

NC = 2
NS = 16
NW = NC * NS

B_PER_W = BATCH // NW
BB = 16
NB = B_PER_W // BB


@functools.partial(
    pl.kernel,
    out_type=jax.ShapeDtypeStruct((BATCH, HIST, N_STATE), jnp.float32),
    mesh=plsc.VectorSubcoreMesh(core_axis_name="c", subcore_axis_name="s"),
    scratch_types=[
        pltpu.VMEM((2, BB, HIST), jnp.int32),
        pltpu.VMEM((2, BB, HIST, N_STATE), jnp.float32),
        pltpu.SemaphoreType.DMA,
        pltpu.SemaphoreType.DMA,
        pltpu.SemaphoreType.DMA,
        pltpu.SemaphoreType.DMA,
    ],
    compiler_params=pltpu.CompilerParams(use_tc_tiling_on_sc=False),
)
def _emb_lookup(idx_hbm, table_hbm, out_hbm, idx_v, rows_v, g0sem, g1sem, w0sem, w1sem):
    wid = lax.axis_index("s") * NC + lax.axis_index("c")
    b0 = wid * B_PER_W
    gsem = (g0sem, g1sem)
    wsem = (w0sem, w1sem)

    def load_idx(t, b):
        pltpu.sync_copy(idx_hbm.at[pl.ds(b0 + t * BB, BB)], idx_v.at[b])

    def fire_gather(b):
        for i in range(BB):
            pltpu.async_copy(table_hbm.at[idx_v.at[b, i]], rows_v.at[b, i], gsem[b])

    def drain_gather(b):
        pltpu.make_async_copy(out_hbm.at[pl.ds(0, BB)], rows_v.at[b], gsem[b]).wait()

    def fire_wb(t, b):
        pltpu.async_copy(rows_v.at[b], out_hbm.at[pl.ds(b0 + t * BB, BB)], wsem[b])

    def wait_wb(b):
        pltpu.make_async_copy(rows_v.at[b], out_hbm.at[pl.ds(0, BB)], wsem[b]).wait()

    load_idx(0, 0)
    fire_gather(0)
    load_idx(1, 1)
    fire_gather(1)
    drain_gather(0)
    fire_wb(0, 0)

    @pl.loop(2, NB, step=2)
    def _steady(t):
        for b in range(2):
            tb = t + b
            load_idx(tb, b)
            wait_wb(b)
            fire_gather(b)
            drain_gather(1 - b)
            fire_wb(tb - 1, 1 - b)

    drain_gather(1)
    fire_wb(NB - 1, 1)
    wait_wb(0)
    wait_wb(1)


def kernel(x, weight):
    return _emb_lookup(x.astype(jnp.int32), weight)


# speedup vs baseline: 2.3902x; 1.2680x over previous
import functools

import jax
import jax.numpy as jnp
from jax import lax
from jax.experimental import pallas as pl
from jax.experimental.pallas import tpu as pltpu
from jax.experimental.pallas import tpu_sc as plsc

N_VOCAB = 1000000
N_STATE = 64
BATCH = 16384
HIST = 50

NC = 2
NS = 16
NW = NC * NS
L = 16

BC = BATCH // 128
N_PATCH = HIST * BC
P_PER_W = N_PATCH // NW


@functools.partial(
    pl.kernel,
    out_type=jax.ShapeDtypeStruct((HIST, 8, BC, 8, 128), jnp.float32),
    mesh=plsc.VectorSubcoreMesh(core_axis_name="c", subcore_axis_name="s"),
    scratch_types=[
        pltpu.VMEM((2, 128), jnp.int32),
        pltpu.VMEM((2, 128, N_STATE), jnp.float32),
        pltpu.VMEM((2, 8, 8, 129), jnp.float32),
        pltpu.SemaphoreType.DMA,
        pltpu.SemaphoreType.DMA,
        pltpu.SemaphoreType.DMA,
        pltpu.SemaphoreType.DMA,
        pltpu.SemaphoreType.DMA,
        pltpu.SemaphoreType.DMA,
    ],
    compiler_params=pltpu.CompilerParams(
        use_tc_tiling_on_sc=False, needs_layout_passes=False
    ),
)
def _emb_lookup(idx_hbm, table_hbm, out_hbm, idx_v, rows_v, obuf,
                i0sem, i1sem, g0sem, g1sem, w0sem, w1sem):
    wid = lax.axis_index("s") * NC + lax.axis_index("c")
    p0 = wid * P_PER_W
    isem = (i0sem, i1sem)
    gsem = (g0sem, g1sem)
    wsem = (w0sem, w1sem)

    def hbc(p):
        pg = p0 + p
        return pg // BC, pg % BC

    def fire_idx(p, s):
        h, bc = hbc(p)
        pltpu.async_copy(idx_hbm.at[h, pl.ds(bc * 128, 128)], idx_v.at[s], isem[s])

    def drain_idx(s):
        pltpu.make_async_copy(idx_hbm.at[0, pl.ds(0, 128)], idx_v.at[s], isem[s]).wait()

    def fire_gather(s):
        pltpu.async_copy(table_hbm.at[idx_v.at[s]], rows_v.at[s], gsem[s])

    def drain_gather(s):
        pltpu.make_async_copy(table_hbm.at[pl.ds(0, 128)], rows_v.at[s], gsem[s]).wait()

    def transpose(s):
        rows = rows_v.at[s]
        ob = obuf.at[s]
        j = lax.iota(jnp.int32, L)
        dsvec = lax.bitwise_and(j, jnp.full((L,), 7, jnp.int32))
        dtvecs = [lax.shift_right_logical(j, 2 + 1) + (2 * k) for k in range(4)]

        @pl.loop(0, 128, unroll=8)
        def _b(b):
            bvec = jnp.full((L,), b, jnp.int32)
            for k in range(4):
                vals = rows[b, pl.ds(L * k, L)]
                plsc.store_scatter(ob, [dtvecs[k], dsvec, bvec], vals)

    def fire_wb(p, s):
        h, bc = hbc(p)
        pltpu.async_copy(
            obuf.at[s, :, :, pl.ds(0, 128)], out_hbm.at[h, :, bc], wsem[s]
        )

    def wait_wb(s):
        pltpu.make_async_copy(
            obuf.at[s, :, :, pl.ds(0, 128)], out_hbm.at[0, :, 0], wsem[s]
        ).wait()

    fire_idx(0, 0)
    fire_idx(1, 1)
    drain_idx(0)
    fire_gather(0)
    drain_idx(1)
    fire_gather(1)
    drain_gather(0)
    fire_idx(2, 0)
    transpose(0)
    fire_wb(0, 0)
    drain_idx(0)
    fire_gather(0)
    drain_gather(1)
    fire_idx(3, 1)
    transpose(1)
    fire_wb(1, 1)

    @pl.loop(2, P_PER_W - 2, step=2)
    def _steady(p):
        for s in range(2):
            pp = p + s
            o = 1 - s
            drain_idx(o)
            fire_gather(o)
            drain_gather(s)
            fire_idx(pp + 2, s)
            wait_wb(s)
            transpose(s)
            fire_wb(pp, s)

    drain_idx(1)
    fire_gather(1)
    drain_gather(0)
    wait_wb(0)
    transpose(0)
    fire_wb(P_PER_W - 2, 0)
    drain_gather(1)
    wait_wb(1)
    transpose(1)
    fire_wb(P_PER_W - 1, 1)
    wait_wb(0)
    wait_wb(1)


def kernel(x, weight):
    xt = x.astype(jnp.int32).T
    out5 = _emb_lookup(xt, weight)
    return out5.transpose((2, 4, 0, 1, 3)).reshape(BATCH, HIST, N_STATE)
